# baseline (speedup 1.0000x reference)
import functools

import jax
import jax.numpy as jnp
from jax import lax
from jax.experimental import pallas as pl
from jax.experimental.pallas import tpu as pltpu
from jax.experimental.pallas import tpu_sc as plsc

_NC = 2
_NS = 16
_NW = _NC * _NS
_L = 16
_B = 128
_N = 128
_M = 128
_BSC = 64
_BTC = _B - _BSC
_BPW = _BSC // _NW
_CH = _M // _L


def _lane_rot(v, idx):
    return jnp.take_along_axis(v, idx, axis=0, mode="promise_in_bounds")


def _dp_body(img_hbm, out_hbm, theta_v, out_v, sems):
    wid = lax.axis_index("s") * _NC + lax.axis_index("c")
    slab = _N * _M
    copies = [
        pltpu.async_copy(
            img_hbm.at[pl.ds((wid * _BPW + b) * slab, slab)],
            theta_v.at[pl.ds(b * slab, slab)],
            sems.at[b],
        )
        for b in range(_BPW)
    ]

    iota = lax.iota(jnp.int32, _L)
    inf_v = jnp.full((_L,), jnp.inf, dtype=jnp.float32)
    rotr_idx = jnp.bitwise_and(iota + (_L - 1), _L - 1)
    rotl_idx = jnp.bitwise_and(iota + 1, _L - 1)
    lane0 = iota == 0
    lane15 = iota == (_L - 1)
    all_lanes = iota >= 0
    offs = [iota * _CH + j for j in range(_CH)]

    for b in range(_BPW):
        copies[b].wait()

        def load_row(i):
            row = theta_v.at[pl.ds((b * _N + i) * _M, _M)]
            return [
                jnp.abs(plsc.load_gather(row, [offs[j]], mask=all_lanes))
                for j in range(_CH)
            ]

        def row_step(i, a):
            th = load_row(i)
            left0 = jnp.where(lane0, inf_v, _lane_rot(a[_CH - 1], rotr_idx))
            right7 = jnp.where(lane15, inf_v, _lane_rot(a[0], rotl_idx))
            new = []
            for j in range(_CH):
                lt = a[j - 1] if j > 0 else left0
                rt = a[j + 1] if j < _CH - 1 else right7
                new.append(th[j] + jnp.minimum(jnp.minimum(lt, a[j]), rt))
            return tuple(new)

        def row_pair(k, a):
            return row_step(2 * k + 2, row_step(2 * k + 1, a))

        a_fin = lax.fori_loop(0, (_N - 2) // 2, row_pair, tuple(load_row(0)))
        a_fin = row_step(_N - 1, a_fin)

        acc = a_fin[0]
        for j in range(1, _CH):
            acc = jnp.minimum(acc, a_fin[j])
        mn = jnp.full((_L,), jnp.min(acc), dtype=jnp.float32)
        plsc.store_scatter(out_v, [jnp.full((_L,), b, jnp.int32)], mn, mask=lane0)

    pltpu.sync_copy(out_v, out_hbm.at[wid])


_TC_RB = 8


def _dp_tc_body(x_ref, o_ref, v_ref):
    i = pl.program_id(0)

    def update(th):
        v = v_ref[...]
        inf_col = jnp.full((_BTC, 1), jnp.inf, dtype=jnp.float32)
        left = jnp.concatenate([inf_col, v[:, :-1]], axis=1)
        right = jnp.concatenate([v[:, 1:], inf_col], axis=1)
        v_ref[...] = th + jnp.minimum(jnp.minimum(left, v), right)

    @pl.when(i == 0)
    def _():
        v_ref[...] = jnp.abs(x_ref[:, 0, :])

    @pl.when(i > 0)
    def _():
        update(jnp.abs(x_ref[:, 0, :]))

    for k in range(1, _TC_RB):
        update(jnp.abs(x_ref[:, k, :]))

    @pl.when(i == pl.num_programs(0) - 1)
    def _():
        o_ref[...] = jnp.min(v_ref[...], axis=1, keepdims=True)


@jax.jit
def kernel(images):
    imgs_flat = images.reshape(_B * _N * _M)
    run_sc = functools.partial(
        pl.kernel,
        out_type=jax.ShapeDtypeStruct((_NW, _BPW), jnp.float32),
        mesh=plsc.VectorSubcoreMesh(core_axis_name="c", subcore_axis_name="s"),
        scratch_types=[
            pltpu.VMEM((_BPW * _N * _M,), jnp.float32),
            pltpu.VMEM((_BPW,), jnp.float32),
            pltpu.SemaphoreType.DMA((_BPW,)),
        ],
        compiler_params=pltpu.CompilerParams(
            needs_layout_passes=False,
            disable_bounds_checks=True,
            disable_semaphore_checks=True,
        ),
    )(_dp_body)
    out_sc = run_sc(imgs_flat)

    out_tc = pl.pallas_call(
        _dp_tc_body,
        grid=(_N // _TC_RB,),
        in_specs=[pl.BlockSpec((_BTC, _TC_RB, _M), lambda i: (1, i, 0))],
        out_specs=pl.BlockSpec((_BTC, 1), lambda i: (0, 0)),
        out_shape=jax.ShapeDtypeStruct((_BTC, 1), jnp.float32),
        scratch_shapes=[pltpu.VMEM((_BTC, _M), jnp.float32)],
    )(images)

    return jnp.concatenate([out_sc.reshape(_BSC), out_tc.reshape(_BTC)])


# speedup vs baseline: 1.2427x; 1.2427x over previous
import functools

import jax
import jax.numpy as jnp
from jax import lax
from jax.experimental import pallas as pl
from jax.experimental.pallas import tpu as pltpu
from jax.experimental.pallas import tpu_sc as plsc

_NC = 2
_NS = 16
_NW = _NC * _NS
_L = 16
_B = 128
_N = 128
_M = 128
_BPW = _B // _NW
_CH = _M // _L


def _lane_rot(v, idx):
    return jnp.take_along_axis(v, idx, axis=0, mode="promise_in_bounds")


def _dp_body(img_hbm, out_hbm, theta_v, out_v, sems):
    wid = lax.axis_index("s") * _NC + lax.axis_index("c")
    slab = _N * _M
    copies = [
        pltpu.async_copy(
            img_hbm.at[pl.ds((wid * _BPW + b) * slab, slab)],
            theta_v.at[pl.ds(b * slab, slab)],
            sems.at[b],
        )
        for b in range(_BPW)
    ]

    iota = lax.iota(jnp.int32, _L)
    inf_v = jnp.full((_L,), jnp.inf, dtype=jnp.float32)
    rotr_idx = jnp.bitwise_and(iota + (_L - 1), _L - 1)
    rotl_idx = jnp.bitwise_and(iota + 1, _L - 1)
    lane0 = iota == 0
    lane15 = iota == (_L - 1)
    all_lanes = iota >= 0
    offs = [iota * _CH + j for j in range(_CH)]

    for b in range(_BPW):
        copies[b].wait()

        def load_row(i):
            row = theta_v.at[pl.ds((b * _N + i) * _M, _M)]
            return [
                jnp.abs(plsc.load_gather(row, [offs[j]], mask=all_lanes))
                for j in range(_CH)
            ]

        def row_step(i, a):
            th = load_row(i)
            left0 = jnp.where(lane0, inf_v, _lane_rot(a[_CH - 1], rotr_idx))
            right7 = jnp.where(lane15, inf_v, _lane_rot(a[0], rotl_idx))
            new = []
            for j in range(_CH):
                lt = a[j - 1] if j > 0 else left0
                rt = a[j + 1] if j < _CH - 1 else right7
                new.append(th[j] + jnp.minimum(jnp.minimum(lt, a[j]), rt))
            return tuple(new)

        def row_pair(k, a):
            return row_step(2 * k + 2, row_step(2 * k + 1, a))

        a_fin = lax.fori_loop(0, (_N - 2) // 2, row_pair, tuple(load_row(0)))
        a_fin = row_step(_N - 1, a_fin)

        acc = a_fin[0]
        for j in range(1, _CH):
            acc = jnp.minimum(acc, a_fin[j])
        mn = jnp.full((_L,), jnp.min(acc), dtype=jnp.float32)
        plsc.store_scatter(out_v, [jnp.full((_L,), b, jnp.int32)], mn, mask=lane0)

    pltpu.sync_copy(out_v, out_hbm.at[wid])


@jax.jit
def kernel(images):
    imgs = images.reshape(_B * _N * _M)
    run = functools.partial(
        pl.kernel,
        out_type=jax.ShapeDtypeStruct((_NW, _BPW), jnp.float32),
        mesh=plsc.VectorSubcoreMesh(core_axis_name="c", subcore_axis_name="s"),
        scratch_types=[
            pltpu.VMEM((_BPW * _N * _M,), jnp.float32),
            pltpu.VMEM((_BPW,), jnp.float32),
            pltpu.SemaphoreType.DMA((_BPW,)),
        ],
        compiler_params=pltpu.CompilerParams(
            needs_layout_passes=False,
            disable_bounds_checks=True,
            disable_semaphore_checks=True,
        ),
    )(_dp_body)
    out = run(imgs)
    return out.reshape(_B)
